# baseline (speedup 1.0000x reference)
import functools

import jax
import jax.numpy as jnp
from jax import lax
from jax.experimental import pallas as pl
from jax.experimental.pallas import tpu as pltpu
from jax.experimental.pallas import tpu_sc as plsc

_N = 1_000_000
_C = 100

_NW = 32
_R = 336
_KCH = 93
_CHUNK = _R * _KCH
_TAILR = _N - (_NW - 1) * _CHUNK - _CHUNK
_HB = 128
_HSIZE = 2 * 16 * _HB


def _sc_scan_body(yp_hbm, yt_hbm, out_hbm, x0, x1, t0, t1, hist_v, sem0, sem1):
    wid = lax.axis_index("s") * 2 + lax.axis_index("c")
    wbase = pl.multiple_of(wid * _CHUNK, 16)

    def _zero(j, _):
        hist_v[pl.ds(j * 16, 16)] = jnp.zeros((16,), jnp.int32)
        return 0

    lax.fori_loop(0, _HSIZE // 16, _zero, 0)

    lanev = lax.iota(jnp.int32, 16)
    ones = jnp.ones((16,), jnp.int32)
    neg = jnp.full((16,), -jnp.inf, jnp.float32)

    def _proc(xbuf, tbuf, nrows):
        def _step(i, _):
            rowv = i * 16 + lanev
            t = tbuf[pl.ds(i * 16, 16)]
            m = neg
            colv = jnp.zeros((16,), jnp.int32)
            for c in range(_C):
                v = plsc.load_gather(xbuf, [rowv, colv])
                m = jnp.maximum(m, v)
                colv = colv + 1
            xv = plsc.load_gather(xbuf, [rowv, t])
            ok = xv >= m
            b = jnp.where(ok, t, _C)
            plsc.addupdate_scatter(hist_v, [lanev * _HB + b], ones)
            plsc.addupdate_scatter(hist_v, [(16 * _HB) + lanev * _HB + t], ones)
            return 0

        lax.fori_loop(0, nrows // 16, _step, 0)

    def _issue(k, xbuf, tbuf, sem):
        base = pl.multiple_of(wbase + k * _R, 16)
        cx = pltpu.make_async_copy(yp_hbm.at[pl.ds(base, _R)], xbuf, sem)
        cx.start()
        ct = pltpu.make_async_copy(yt_hbm.at[pl.ds(base, _R)], tbuf, sem)
        ct.start()
        return cx, ct

    def _wait(xbuf, tbuf, sem):
        pltpu.make_async_copy(yp_hbm.at[pl.ds(0, _R)], xbuf, sem).wait()
        pltpu.make_async_copy(yt_hbm.at[pl.ds(0, _R)], tbuf, sem).wait()

    _issue(0, x0, t0, sem0)

    def _pair(p, _):
        k = p * 2
        @pl.when(k + 1 < _KCH)
        def _():
            _issue(k + 1, x1, t1, sem1)

        _wait(x0, t0, sem0)
        _proc(x0, t0, _R)

        @pl.when(k + 2 < _KCH)
        def _():
            _issue(k + 2, x0, t0, sem0)

        @pl.when(k + 1 < _KCH)
        def _():
            _wait(x1, t1, sem1)
            _proc(x1, t1, _R)

        return 0

    lax.fori_loop(0, (_KCH + 1) // 2, _pair, 0)

    @pl.when(wid == _NW - 1)
    def _tail():
        base = pl.multiple_of(_N - _TAILR, 16)
        cx = pltpu.make_async_copy(
            yp_hbm.at[pl.ds(base, _TAILR)], x0.at[pl.ds(0, _TAILR)], sem0
        )
        cx.start()
        cx.wait()
        ct = pltpu.make_async_copy(
            yt_hbm.at[pl.ds(base, _TAILR)], t0.at[pl.ds(0, _TAILR)], sem0
        )
        ct.start()
        ct.wait()
        _proc(x0, t0, _TAILR)

    pltpu.sync_copy(hist_v, out_hbm.at[wid])


def _fin_body(h_ref, w_ref, out_ref):
    h = h_ref[...]
    s = jnp.sum(h, axis=0, keepdims=True)
    cp = jnp.zeros((1, _HB), jnp.int32)
    ct = jnp.zeros((1, _HB), jnp.int32)
    for l in range(16):
        cp = cp + s[0:1, l * _HB : (l + 1) * _HB]
        ct = ct + s[0:1, 16 * _HB + l * _HB : 16 * _HB + (l + 1) * _HB]
    ctc = ct[0:1, :_C]
    acc = jnp.where(
        ctc > 0,
        cp[0:1, :_C].astype(jnp.float32) / jnp.maximum(ctc, 1).astype(jnp.float32),
        0.0,
    )
    w = w_ref[...]
    val = jnp.sum(acc * w) / jnp.sum(w)
    out_ref[...] = jnp.broadcast_to(val, (1, 1))


def kernel(y_pred, y_true, weights):
    yt32 = y_true.astype(jnp.int32)

    sc_scan = functools.partial(
        pl.kernel,
        mesh=plsc.VectorSubcoreMesh(core_axis_name="c", subcore_axis_name="s"),
        out_type=jax.ShapeDtypeStruct((_NW, _HSIZE), jnp.int32),
        scratch_types=[
            pltpu.VMEM((_R, _C), jnp.float32),
            pltpu.VMEM((_R, _C), jnp.float32),
            pltpu.VMEM((_R,), jnp.int32),
            pltpu.VMEM((_R,), jnp.int32),
            pltpu.VMEM((_HSIZE,), jnp.int32),
            pltpu.SemaphoreType.DMA,
            pltpu.SemaphoreType.DMA,
        ],
        compiler_params=pltpu.CompilerParams(needs_layout_passes=False),
    )(_sc_scan_body)
    hists = sc_scan(y_pred, yt32)

    out = pl.pallas_call(
        _fin_body,
        in_specs=[
            pl.BlockSpec((_NW, _HSIZE), lambda: (0, 0)),
            pl.BlockSpec((1, _C), lambda: (0, 0)),
        ],
        out_specs=pl.BlockSpec((1, 1), lambda: (0, 0)),
        out_shape=jax.ShapeDtypeStruct((1, 1), jnp.float32),
    )(hists, weights.reshape(1, _C))
    return out.reshape(())


# speedup vs baseline: 1.9614x; 1.9614x over previous
import functools

import jax
import jax.numpy as jnp
from jax import lax
from jax.experimental import pallas as pl
from jax.experimental.pallas import tpu as pltpu
from jax.experimental.pallas import tpu_sc as plsc

_N = 1_000_000
_C = 100
_B = 2048
_GRID = -(-_N // _B)

_NW = 32
_CHUNK = 31248
_TAIL = _N - (_NW - 1) * _CHUNK
_STEPS = _CHUNK // 16
_TSTEPS = _TAIL // 16
_HB = 128
_HSIZE = 2 * 16 * _HB


def _amax_body(yp_ref, yt_ref, out_ref):
    x = yp_ref[...]
    xt = jnp.swapaxes(x, 0, 1)
    ytv = yt_ref[...][None, :]
    idxs = lax.broadcasted_iota(jnp.int32, (_C, _B), 0)
    neg = jnp.float32(-jnp.inf)
    xv = jnp.max(jnp.where(idxs == ytv, xt, neg), axis=0, keepdims=True)
    m = jnp.max(xt, axis=0, keepdims=True)
    correct = xv >= m
    out_ref[...] = jnp.where(correct, ytv, _C).astype(jnp.int32)[0]


def _sc_hist_body(bin_hbm, yt_hbm, out_hbm, bin_v, yt_v, hist_v):
    wid = lax.axis_index("s") * 2 + lax.axis_index("c")
    base = pl.multiple_of(wid * _CHUNK, 16)

    def _zero(j, _):
        hist_v[pl.ds(j * 16, 16)] = jnp.zeros((16,), jnp.int32)
        return 0

    lax.fori_loop(0, _HSIZE // 16, _zero, 0)

    pltpu.sync_copy(bin_hbm.at[pl.ds(base, _CHUNK)], bin_v.at[pl.ds(0, _CHUNK)])
    pltpu.sync_copy(yt_hbm.at[pl.ds(base, _CHUNK)], yt_v.at[pl.ds(0, _CHUNK)])

    @pl.when(wid == _NW - 1)
    def _tail_copy():
        off = _N - (_TAIL - _CHUNK)
        pltpu.sync_copy(
            bin_hbm.at[pl.ds(off, _TAIL - _CHUNK)],
            bin_v.at[pl.ds(_CHUNK, _TAIL - _CHUNK)],
        )
        pltpu.sync_copy(
            yt_hbm.at[pl.ds(off, _TAIL - _CHUNK)],
            yt_v.at[pl.ds(_CHUNK, _TAIL - _CHUNK)],
        )

    lanes = lax.iota(jnp.int32, 16) * _HB
    ones = jnp.ones((16,), jnp.int32)

    def _step(i, _):
        b = bin_v[pl.ds(i * 16, 16)]
        t = yt_v[pl.ds(i * 16, 16)]
        plsc.addupdate_scatter(hist_v, [lanes + b], ones)
        plsc.addupdate_scatter(hist_v, [(16 * _HB) + lanes + t], ones)
        return 0

    lax.fori_loop(0, _STEPS, _step, 0)

    @pl.when(wid == _NW - 1)
    def _tail_steps():
        lax.fori_loop(_STEPS, _TSTEPS, _step, 0)

    pltpu.sync_copy(hist_v, out_hbm.at[wid])


def _fin_body(h_ref, w_ref, out_ref):
    h = h_ref[...]
    s = jnp.sum(h, axis=0, keepdims=True)
    cp = jnp.zeros((1, _HB), jnp.int32)
    ct = jnp.zeros((1, _HB), jnp.int32)
    for l in range(16):
        cp = cp + s[0:1, l * _HB : (l + 1) * _HB]
        ct = ct + s[0:1, 16 * _HB + l * _HB : 16 * _HB + (l + 1) * _HB]
    ctc = ct[0:1, :_C]
    acc = jnp.where(
        ctc > 0,
        cp[0:1, :_C].astype(jnp.float32) / jnp.maximum(ctc, 1).astype(jnp.float32),
        0.0,
    )
    w = w_ref[...]
    val = jnp.sum(acc * w) / jnp.sum(w)
    out_ref[...] = jnp.broadcast_to(val, (1, 1))


def kernel(y_pred, y_true, weights):
    yt32 = y_true.astype(jnp.int32)

    masked_bin = pl.pallas_call(
        _amax_body,
        grid=(_GRID,),
        in_specs=[
            pl.BlockSpec((_B, _C), lambda i: (i, 0)),
            pl.BlockSpec((_B,), lambda i: (i,)),
        ],
        out_specs=pl.BlockSpec((_B,), lambda i: (i,)),
        out_shape=jax.ShapeDtypeStruct((_N,), jnp.int32),
    )(y_pred, yt32)

    sc_hist = functools.partial(
        pl.kernel,
        mesh=plsc.VectorSubcoreMesh(core_axis_name="c", subcore_axis_name="s"),
        out_type=jax.ShapeDtypeStruct((_NW, _HSIZE), jnp.int32),
        scratch_types=[
            pltpu.VMEM((_TAIL,), jnp.int32),
            pltpu.VMEM((_TAIL,), jnp.int32),
            pltpu.VMEM((_HSIZE,), jnp.int32),
        ],
        compiler_params=pltpu.CompilerParams(needs_layout_passes=False),
    )(_sc_hist_body)
    hists = sc_hist(masked_bin, yt32)

    out = pl.pallas_call(
        _fin_body,
        in_specs=[
            pl.BlockSpec((_NW, _HSIZE), lambda: (0, 0)),
            pl.BlockSpec((1, _C), lambda: (0, 0)),
        ],
        out_specs=pl.BlockSpec((1, 1), lambda: (0, 0)),
        out_shape=jax.ShapeDtypeStruct((1, 1), jnp.float32),
    )(hists, weights.reshape(1, _C))
    return out.reshape(())
